# Initial kernel scaffold; baseline (speedup 1.0000x reference)
import jax
import jax.numpy as jnp
from jax.experimental import pallas as pl


def kernel(x, edge_index, batch_index, W1, W2, W_skip):
    raise NotImplementedError("write your pallas kernel here")


# speedup vs baseline: 13.5683x; 13.5683x over previous
import functools

import jax
import jax.numpy as jnp
from jax import lax
from jax.experimental import pallas as pl
from jax.experimental.pallas import tpu as pltpu
from jax.experimental.pallas import tpu_sc as plsc

NC = 2
NS = 16
NW = NC * NS
CH = 128


def _cdiv(a, b):
    return (a + b - 1) // b


def _make_deg_kernel(nch, ndeg):
    stripe = ndeg // NS
    mesh = plsc.VectorSubcoreMesh(core_axis_name="c", subcore_axis_name="s")

    @functools.partial(
        pl.kernel,
        out_type=jax.ShapeDtypeStruct((NC, ndeg), jnp.float32),
        mesh=mesh,
        scratch_types=[
            pltpu.VMEM((nch, CH), jnp.int32),
            pltpu.VMEM((CH,), jnp.float32),
            pltpu.VMEM((stripe,), jnp.float32),
            pltpu.VMEM_SHARED((ndeg,), jnp.float32),
        ],
    )
    def deg_kernel(dst_hbm, out_hbm, dst_v, ones_v, zbuf_v, acc_sh):
        c = lax.axis_index("c")
        s = lax.axis_index("s")
        wid = c * NS + s

        def fill_ones(i, carry):
            ones_v[pl.ds(i * 16, 16)] = jnp.ones((16,), jnp.float32)
            return carry
        lax.fori_loop(0, CH // 16, fill_ones, 0)

        def fill_zeros(i, carry):
            zbuf_v[pl.ds(i * 16, 16)] = jnp.zeros((16,), jnp.float32)
            return carry
        lax.fori_loop(0, stripe // 16, fill_zeros, 0)

        pltpu.sync_copy(zbuf_v, acc_sh.at[pl.ds(s * stripe, stripe)])
        pltpu.sync_copy(dst_hbm.at[wid], dst_v)
        plsc.subcore_barrier()

        def body(j, carry):
            pltpu.sync_copy(ones_v, acc_sh.at[dst_v.at[j]], add=True)
            return carry
        lax.fori_loop(0, nch, body, 0)

        plsc.subcore_barrier()
        pltpu.sync_copy(acc_sh.at[pl.ds(s * stripe, stripe)],
                        out_hbm.at[c].at[pl.ds(s * stripe, stripe)])

    return deg_kernel


def _make_scatter_kernel(d, nch, nacc):
    stripe = nacc // NS
    mesh = plsc.VectorSubcoreMesh(core_axis_name="c", subcore_axis_name="s")

    @functools.partial(
        pl.kernel,
        out_type=jax.ShapeDtypeStruct((NC, nacc, d), jnp.float32),
        mesh=mesh,
        scratch_types=[
            pltpu.VMEM((nch, CH), jnp.int32),
            pltpu.VMEM((nch, CH), jnp.int32),
            pltpu.VMEM((CH, d), jnp.float32),
            pltpu.VMEM_SHARED((nacc, d), jnp.float32),
            pltpu.SemaphoreType.DMA,
        ],
    )
    def sc_kernel(table_hbm, src_hbm, dst_hbm, out_hbm,
                  src_v, dst_v, gbuf, acc_sh, sem):
        c = lax.axis_index("c")
        s = lax.axis_index("s")
        wid = c * NS + s

        def zrow(i, carry):
            def zcol(k, carry2):
                gbuf[i, pl.ds(k * 16, 16)] = jnp.zeros((16,), jnp.float32)
                return carry2
            lax.fori_loop(0, d // 16, zcol, 0)
            return carry
        lax.fori_loop(0, CH, zrow, 0)

        off = 0
        while off < stripe:
            ln = min(CH, stripe - off)
            pltpu.sync_copy(gbuf.at[pl.ds(0, ln)],
                            acc_sh.at[pl.ds(s * stripe + off, ln)])
            off += ln
        pltpu.sync_copy(src_hbm.at[wid], src_v)
        pltpu.sync_copy(dst_hbm.at[wid], dst_v)
        plsc.subcore_barrier()

        def body(j, carry):
            pltpu.async_copy(table_hbm.at[src_v.at[j]], gbuf, sem).wait()
            pltpu.sync_copy(gbuf, acc_sh.at[dst_v.at[j]], add=True)
            return carry
        lax.fori_loop(0, nch, body, 0)

        plsc.subcore_barrier()
        pltpu.sync_copy(acc_sh.at[pl.ds(s * stripe, stripe)],
                        out_hbm.at[c].at[pl.ds(s * stripe, stripe)])

    return sc_kernel


def _tc_stage1(degT, x, W1T, WskipT, rb):
    n, d_in = x.shape
    d_h = W1T.shape[1]
    d_c = WskipT.shape[1]
    nb = n // rb

    def body(deg_ref, x_ref, w1_ref, ws_ref, h1s_ref, hs_ref, dinv_ref):
        deg = deg_ref[...]
        dinv = lax.rsqrt(deg[:, 0:1] + deg[:, 1:2] + 1.0)
        xb = x_ref[...]
        h1s_ref[...] = dinv * jnp.dot(xb, w1_ref[...],
                                      preferred_element_type=jnp.float32)
        hs_ref[...] = jnp.dot(xb, ws_ref[...],
                              preferred_element_type=jnp.float32)
        dinv_ref[...] = dinv

    return pl.pallas_call(
        body,
        grid=(nb,),
        in_specs=[
            pl.BlockSpec((rb, 2), lambda i: (i, 0)),
            pl.BlockSpec((rb, d_in), lambda i: (i, 0)),
            pl.BlockSpec((d_in, d_h), lambda i: (0, 0)),
            pl.BlockSpec((d_in, d_c), lambda i: (0, 0)),
        ],
        out_specs=[
            pl.BlockSpec((rb, d_h), lambda i: (i, 0)),
            pl.BlockSpec((rb, d_c), lambda i: (i, 0)),
            pl.BlockSpec((rb, 1), lambda i: (i, 0)),
        ],
        out_shape=[
            jax.ShapeDtypeStruct((n, d_h), jnp.float32),
            jax.ShapeDtypeStruct((n, d_c), jnp.float32),
            jax.ShapeDtypeStruct((n, 1), jnp.float32),
        ],
    )(degT, x, W1T, WskipT)


def _tc_stage2(P0, P1, h1s, dinv, rb):
    n, d_h = h1s.shape
    nb = n // rb

    def body(p0_ref, p1_ref, h1s_ref, dinv_ref, u_ref):
        dinv = dinv_ref[...]
        agg = p0_ref[...] + p1_ref[...] + h1s_ref[...]
        u_ref[...] = dinv * jnp.maximum(dinv * agg, 0.0)

    return pl.pallas_call(
        body,
        grid=(nb,),
        in_specs=[
            pl.BlockSpec((rb, d_h), lambda i: (i, 0)),
            pl.BlockSpec((rb, d_h), lambda i: (i, 0)),
            pl.BlockSpec((rb, d_h), lambda i: (i, 0)),
            pl.BlockSpec((rb, 1), lambda i: (i, 0)),
        ],
        out_specs=pl.BlockSpec((rb, d_h), lambda i: (i, 0)),
        out_shape=jax.ShapeDtypeStruct((n, d_h), jnp.float32),
    )(P0, P1, h1s, dinv)


def _tc_stage3(U0, U1, u, dinv, Hs, W2T, rb):
    n, d_h = u.shape
    d_c = W2T.shape[1]
    nb = n // rb

    def body(u0_ref, u1_ref, u_ref, dinv_ref, hs_ref, w2_ref, hid_ref, out_ref):
        dinv = dinv_ref[...]
        m = u0_ref[...] + u1_ref[...] + u_ref[...]
        hidden2 = dinv * jnp.dot(m, w2_ref[...],
                                 preferred_element_type=jnp.float32)
        hid_ref[...] = hidden2
        z = hidden2 + hs_ref[...]
        z = z - jnp.max(z, axis=1, keepdims=True)
        e = jnp.exp(z)
        out_ref[...] = e / jnp.sum(e, axis=1, keepdims=True)

    return pl.pallas_call(
        body,
        grid=(nb,),
        in_specs=[
            pl.BlockSpec((rb, d_h), lambda i: (i, 0)),
            pl.BlockSpec((rb, d_h), lambda i: (i, 0)),
            pl.BlockSpec((rb, d_h), lambda i: (i, 0)),
            pl.BlockSpec((rb, 1), lambda i: (i, 0)),
            pl.BlockSpec((rb, d_c), lambda i: (i, 0)),
            pl.BlockSpec((d_h, d_c), lambda i: (0, 0)),
        ],
        out_specs=[
            pl.BlockSpec((rb, d_c), lambda i: (i, 0)),
            pl.BlockSpec((rb, d_c), lambda i: (i, 0)),
        ],
        out_shape=[
            jax.ShapeDtypeStruct((n, d_c), jnp.float32),
            jax.ShapeDtypeStruct((n, d_c), jnp.float32),
        ],
    )(U0, U1, u, dinv, Hs, W2T)


def kernel(x, edge_index, batch_index, W1, W2, W_skip):
    n, d_in = x.shape
    e = edge_index.shape[1]
    d_h = W1.shape[0]
    d_c = W2.shape[0]

    nch = _cdiv(_cdiv(e, NW), CH)
    tot = NW * nch * CH
    src = jnp.concatenate(
        [edge_index[0], jnp.zeros((tot - e,), jnp.int32)]).reshape(NW, nch, CH)
    dst = jnp.concatenate(
        [edge_index[1], jnp.full((tot - e,), n, jnp.int32)]).reshape(NW, nch, CH)

    ndeg = ((n + 1 + NS * 128 - 1) // (NS * 128)) * (NS * 128)
    nacc = ((n + 1 + NS * 8 - 1) // (NS * 8)) * (NS * 8)

    degs = _make_deg_kernel(nch, ndeg)(dst)
    degT = jnp.stack([degs[0, :n], degs[1, :n]], axis=1)

    rb = n // 10 if (n % 10 == 0 and (n // 10) % 8 == 0) else 8
    h1s, Hs, dinv = _tc_stage1(degT, x, W1.T, W_skip.T, rb)

    scatter = _make_scatter_kernel(d_h, nch, nacc)
    P = scatter(h1s, src, dst)
    u = _tc_stage2(P[0, :n], P[1, :n], h1s, dinv, rb)

    U = scatter(u, src, dst)
    hidden2, output = _tc_stage3(U[0, :n], U[1, :n], u, dinv, Hs, W2.T, rb)
    return (hidden2, output)
